# baseline (speedup 1.0000x reference)
import functools

import jax
import jax.numpy as jnp
from jax import lax
from jax.experimental import pallas as pl
from jax.experimental.pallas import tpu as pltpu
from jax.experimental.pallas import tpu_sc as plsc

N = 10000
E = 320000
C = 128
CHUNK = 128
NTILES = 32
RPS = 632
RPS_LAST = N - 15 * RPS


def _mesh():
    return plsc.VectorSubcoreMesh(core_axis_name="c", subcore_axis_name="s",
                                  num_cores=2)


def _per_subcore_slice(sid, fn):
    lo = pl.multiple_of(sid * RPS, 8)

    @pl.when(sid < 15)
    def _():
        fn(lo, RPS)

    @pl.when(sid == 15)
    def _():
        fn(lo, RPS_LAST)


def _sc_degree(dst, ones_chunk, zrows):
    nch = dst.shape[0] // CHUNK

    @functools.partial(
        pl.kernel,
        out_type=jax.ShapeDtypeStruct((2, N, C), jnp.float32),
        mesh=_mesh(),
        scratch_types=[
            pltpu.VMEM((1, CHUNK), jnp.int32),
            pltpu.VMEM((CHUNK, C), jnp.float32),
            pltpu.VMEM_SHARED((N, C), jnp.float32),
        ],
    )
    def k(dst_hbm, ones_hbm, z_hbm, out_hbm, idx_v, ones_v, acc_sh):
        cid = lax.axis_index("c")
        sid = lax.axis_index("s")
        wid = sid * 2 + cid
        pltpu.sync_copy(ones_hbm, ones_v)
        _per_subcore_slice(sid, lambda lo, n: pltpu.sync_copy(
            z_hbm.at[pl.ds(lo, n)], acc_sh.at[pl.ds(lo, n)]))
        plsc.subcore_barrier()

        @pl.loop(wid, nch, step=NTILES)
        def _(g):
            pltpu.sync_copy(dst_hbm.at[pl.ds(g * CHUNK, CHUNK)], idx_v.at[0])
            pltpu.sync_copy(ones_v, acc_sh.at[idx_v.at[0]], add=True)

        plsc.subcore_barrier()
        _per_subcore_slice(sid, lambda lo, n: pltpu.sync_copy(
            acc_sh.at[pl.ds(lo, n)], out_hbm.at[cid, pl.ds(lo, n)]))

    return k(dst, ones_chunk, zrows)


SUP = 2
SUPE = SUP * CHUNK


def _sc_aggregate(y, src, dst, zrows):
    nch = src.shape[0] // CHUNK

    @functools.partial(
        pl.kernel,
        out_type=jax.ShapeDtypeStruct((2, N, C), jnp.float32),
        mesh=_mesh(),
        scratch_types=[
            pltpu.VMEM((1, CHUNK), jnp.int32),
            pltpu.VMEM((1, CHUNK), jnp.int32),
            pltpu.VMEM((CHUNK, C), jnp.float32),
            pltpu.VMEM_SHARED((N, C), jnp.float32),
        ],
    )
    def k(y_hbm, src_hbm, dst_hbm, z_hbm, out_hbm, src_v, dst_v, rows_v,
          acc_sh):
        cid = lax.axis_index("c")
        sid = lax.axis_index("s")
        wid = sid * 2 + cid
        _per_subcore_slice(sid, lambda lo, n: pltpu.sync_copy(
            z_hbm.at[pl.ds(lo, n)], acc_sh.at[pl.ds(lo, n)]))
        plsc.subcore_barrier()

        @pl.loop(wid, nch, step=NTILES)
        def _(g):
            base = g * CHUNK
            pltpu.sync_copy(src_hbm.at[pl.ds(base, CHUNK)], src_v.at[0])
            pltpu.sync_copy(dst_hbm.at[pl.ds(base, CHUNK)], dst_v.at[0])
            pltpu.sync_copy(y_hbm.at[src_v.at[0]], rows_v)
            pltpu.sync_copy(rows_v, acc_sh.at[dst_v.at[0]], add=True)

        plsc.subcore_barrier()
        _per_subcore_slice(sid, lambda lo, n: pltpu.sync_copy(
            acc_sh.at[pl.ds(lo, n)], out_hbm.at[cid, pl.ds(lo, n)]))

    return k(y, src, dst, zrows)


def _sc_gather(table, idx):
    nidx = idx.shape[0]
    width = table.shape[1]

    @functools.partial(
        pl.kernel,
        out_type=jax.ShapeDtypeStruct((nidx, width), table.dtype),
        mesh=_mesh(),
        compiler_params=pltpu.CompilerParams(use_tc_tiling_on_sc=False),
        scratch_types=[
            pltpu.VMEM((1, CHUNK), jnp.int32),
            pltpu.VMEM((CHUNK, width), table.dtype),
        ],
    )
    def k(t_hbm, i_hbm, o_hbm, idx_v, rows_v):
        cid = lax.axis_index("c")
        sid = lax.axis_index("s")
        wid = sid * 2 + cid

        @pl.loop(wid, nidx // CHUNK, step=NTILES)
        def _(g):
            base = g * CHUNK
            pltpu.sync_copy(i_hbm.at[pl.ds(base, CHUNK)], idx_v.at[0])
            pltpu.sync_copy(t_hbm.at[idx_v.at[0]], rows_v)
            pltpu.sync_copy(rows_v, o_hbm.at[pl.ds(base, CHUNK)])

    return k(table, idx)


_BN = 1000
_BE = 2000


def _tc_prep(deg2, x):
    def body(d0, d1, xb, dinv_ref, y1_ref):
        deg = d0[0][:, 0:1] + d1[0][:, 0:1] + 1.0
        dinv = lax.rsqrt(jnp.maximum(deg, 1.0))
        dinv_ref[...] = dinv
        y1_ref[...] = xb[...] * dinv

    return pl.pallas_call(
        body,
        grid=(N // _BN,),
        in_specs=[
            pl.BlockSpec((1, _BN, C), lambda i: (0, i, 0)),
            pl.BlockSpec((1, _BN, C), lambda i: (1, i, 0)),
            pl.BlockSpec((_BN, C), lambda i: (i, 0)),
        ],
        out_specs=[
            pl.BlockSpec((_BN, 1), lambda i: (i, 0)),
            pl.BlockSpec((_BN, C), lambda i: (i, 0)),
        ],
        out_shape=[
            jax.ShapeDtypeStruct((N, 1), jnp.float32),
            jax.ShapeDtypeStruct((N, C), jnp.float32),
        ],
    )(deg2, deg2, x)


def _tc_mid(p, y1, dinv, W1, b1, W2):
    def body(a0, a1, y1b, dv, w1, b1b, w2, y2_ref):
        a = (a0[0] + a1[0] + y1b[...]) * dv[...]
        z1 = jnp.maximum(
            jnp.dot(a, w1[...], preferred_element_type=jnp.float32) + b1b[...],
            0.0)
        h2 = jnp.dot(z1, w2[...], preferred_element_type=jnp.float32)
        y2_ref[...] = h2 * dv[...]

    return pl.pallas_call(
        body,
        grid=(N // _BN,),
        in_specs=[
            pl.BlockSpec((1, _BN, C), lambda i: (0, i, 0)),
            pl.BlockSpec((1, _BN, C), lambda i: (1, i, 0)),
            pl.BlockSpec((_BN, C), lambda i: (i, 0)),
            pl.BlockSpec((_BN, 1), lambda i: (i, 0)),
            pl.BlockSpec((C, 2 * C), lambda i: (0, 0)),
            pl.BlockSpec((1, 2 * C), lambda i: (0, 0)),
            pl.BlockSpec((2 * C, C), lambda i: (0, 0)),
        ],
        out_specs=pl.BlockSpec((_BN, C), lambda i: (i, 0)),
        out_shape=jax.ShapeDtypeStruct((N, C), jnp.float32),
    )(p, p, y1, dinv, W1, b1, W2)


def _tc_table(p, y2, dinv, b2, fc1Ws, fc1bs):
    def body(a0, a1, y2b, dv, b2b, w, bb, out_ref):
        z = (a0[0] + a1[0] + y2b[...]) * dv[...] + b2b[...]
        p = jnp.dot(z, w[0], preferred_element_type=jnp.float32) + bb[0]
        out_ref[0] = p.astype(jnp.bfloat16)

    return pl.pallas_call(
        body,
        grid=(N // _BN, 2),
        in_specs=[
            pl.BlockSpec((1, _BN, C), lambda i, j: (0, i, 0)),
            pl.BlockSpec((1, _BN, C), lambda i, j: (1, i, 0)),
            pl.BlockSpec((_BN, C), lambda i, j: (i, 0)),
            pl.BlockSpec((_BN, 1), lambda i, j: (i, 0)),
            pl.BlockSpec((1, C), lambda i, j: (0, 0)),
            pl.BlockSpec((1, C, C), lambda i, j: (j, 0, 0)),
            pl.BlockSpec((1, 1, C), lambda i, j: (j, 0, 0)),
        ],
        out_specs=pl.BlockSpec((1, _BN, C), lambda i, j: (j, i, 0)),
        out_shape=jax.ShapeDtypeStruct((2, N, C), jnp.bfloat16),
    )(p, p, y2, dinv, b2, fc1Ws, fc1bs)


def _tc_decoder(G, fc2_W, fc2_b, fc3_W, fc3_b, w4row, b4):
    nb = E // _BE

    def body(g0, g1, w2, b2b, w3, b3b, w4, b4b, o_ref):
        u = jnp.maximum(g0[...].astype(jnp.float32) +
                        g1[...].astype(jnp.float32), 0.0)
        h1 = jnp.maximum(
            jnp.dot(u, w2[...], preferred_element_type=jnp.float32) + b2b[...],
            0.0)
        h2 = jnp.maximum(
            jnp.dot(h1, w3[...], preferred_element_type=jnp.float32) + b3b[...],
            0.0)
        o_ref[...] = jnp.sum(h2 * w4[...], axis=1, keepdims=True) + b4b[...]

    return pl.pallas_call(
        body,
        grid=(nb,),
        in_specs=[
            pl.BlockSpec((_BE, C), lambda i: (i, 0)),
            pl.BlockSpec((_BE, C), lambda i: (i + nb, 0)),
            pl.BlockSpec((C, 64), lambda i: (0, 0)),
            pl.BlockSpec((1, 64), lambda i: (0, 0)),
            pl.BlockSpec((64, 32), lambda i: (0, 0)),
            pl.BlockSpec((1, 32), lambda i: (0, 0)),
            pl.BlockSpec((1, 32), lambda i: (0, 0)),
            pl.BlockSpec((1, 1), lambda i: (0, 0)),
        ],
        out_specs=pl.BlockSpec((_BE, 1), lambda i: (i, 0)),
        out_shape=jax.ShapeDtypeStruct((E, 1), jnp.float32),
    )(G, G, fc2_W, fc2_b, fc3_W, fc3_b, w4row, b4)


def kernel(x, edge_index, W1, b1, W2, b2, fc1_W, fc1_b, fc2_W, fc2_b,
           fc3_W, fc3_b, fc4_W, fc4_b):
    e0 = edge_index[0].astype(jnp.int32)
    e1 = edge_index[1].astype(jnp.int32)

    ones_chunk = jnp.ones((CHUNK, C), jnp.float32)
    zerosNC = jnp.zeros((N, C), jnp.float32)

    deg2 = _sc_degree(e1, ones_chunk, zerosNC)
    dinv, y1 = _tc_prep(deg2, x)
    p1 = _sc_aggregate(y1, e0, e1, zerosNC)
    y2 = _tc_mid(p1, y1, dinv, W1, b1.reshape(1, -1), W2)
    p2 = _sc_aggregate(y2, e0, e1, zerosNC)

    fc1Ws = fc1_W.reshape(2, C, C)
    fc1bs = jnp.stack([fc1_b, jnp.zeros_like(fc1_b)]).reshape(2, 1, C)
    table = _tc_table(p2, y2, dinv, b2.reshape(1, -1), fc1Ws, fc1bs)

    idx_cat = jnp.concatenate([e0, e1 + N])
    t32 = lax.bitcast_convert_type(table.reshape(2 * N, C // 2, 2),
                                   jnp.float32)
    G32 = _sc_gather(t32, idx_cat)
    G = lax.bitcast_convert_type(G32, jnp.bfloat16).reshape(2 * E, C)

    out2d = _tc_decoder(G, fc2_W, fc2_b.reshape(1, -1), fc3_W,
                        fc3_b.reshape(1, -1), fc4_W.reshape(1, -1),
                        fc4_b.reshape(1, 1))
    return out2d.reshape(E)


# speedup vs baseline: 2.2293x; 2.2293x over previous
import functools

import jax
import jax.numpy as jnp
from jax import lax
from jax.experimental import pallas as pl
from jax.experimental.pallas import tpu as pltpu
from jax.experimental.pallas import tpu_sc as plsc

N = 10000
E = 320000
C = 128
CHUNK = 128
NTILES = 32
RPS = 632
RPS_LAST = N - 15 * RPS


def _mesh():
    return plsc.VectorSubcoreMesh(core_axis_name="c", subcore_axis_name="s",
                                  num_cores=2)


def _per_subcore_slice(sid, fn):
    lo = pl.multiple_of(sid * RPS, 8)

    @pl.when(sid < 15)
    def _():
        fn(lo, RPS)

    @pl.when(sid == 15)
    def _():
        fn(lo, RPS_LAST)


def _sc_degree(dst, ones_chunk, zrows):
    nch = dst.shape[0] // CHUNK

    @functools.partial(
        pl.kernel,
        out_type=jax.ShapeDtypeStruct((2, N, C), jnp.float32),
        mesh=_mesh(),
        scratch_types=[
            pltpu.VMEM((1, CHUNK), jnp.int32),
            pltpu.VMEM((CHUNK, C), jnp.float32),
            pltpu.VMEM_SHARED((N, C), jnp.float32),
        ],
    )
    def k(dst_hbm, ones_hbm, z_hbm, out_hbm, idx_v, ones_v, acc_sh):
        cid = lax.axis_index("c")
        sid = lax.axis_index("s")
        wid = sid * 2 + cid
        pltpu.sync_copy(ones_hbm, ones_v)
        _per_subcore_slice(sid, lambda lo, n: pltpu.sync_copy(
            z_hbm.at[pl.ds(lo, n)], acc_sh.at[pl.ds(lo, n)]))
        plsc.subcore_barrier()

        @pl.loop(wid, nch, step=NTILES)
        def _(g):
            pltpu.sync_copy(dst_hbm.at[pl.ds(g * CHUNK, CHUNK)], idx_v.at[0])
            pltpu.sync_copy(ones_v, acc_sh.at[idx_v.at[0]], add=True)

        plsc.subcore_barrier()
        _per_subcore_slice(sid, lambda lo, n: pltpu.sync_copy(
            acc_sh.at[pl.ds(lo, n)], out_hbm.at[cid, pl.ds(lo, n)]))

    return k(dst, ones_chunk, zrows)


SUP = 2
SUPE = SUP * CHUNK


def _sc_aggregate(y, src, dst, zrows):
    nch = src.shape[0] // CHUNK

    @functools.partial(
        pl.kernel,
        out_type=jax.ShapeDtypeStruct((2, N, C), jnp.float32),
        mesh=_mesh(),
        scratch_types=[
            pltpu.VMEM((1, CHUNK), jnp.int32),
            pltpu.VMEM((1, CHUNK), jnp.int32),
            pltpu.VMEM((CHUNK, C), jnp.float32),
            pltpu.VMEM_SHARED((N, C), jnp.float32),
        ],
    )
    def k(y_hbm, src_hbm, dst_hbm, z_hbm, out_hbm, src_v, dst_v, rows_v,
          acc_sh):
        cid = lax.axis_index("c")
        sid = lax.axis_index("s")
        wid = sid * 2 + cid
        _per_subcore_slice(sid, lambda lo, n: pltpu.sync_copy(
            z_hbm.at[pl.ds(lo, n)], acc_sh.at[pl.ds(lo, n)]))
        plsc.subcore_barrier()

        @pl.loop(wid, nch, step=NTILES)
        def _(g):
            base = g * CHUNK
            pltpu.sync_copy(src_hbm.at[pl.ds(base, CHUNK)], src_v.at[0])
            pltpu.sync_copy(dst_hbm.at[pl.ds(base, CHUNK)], dst_v.at[0])
            pltpu.sync_copy(y_hbm.at[src_v.at[0]], rows_v)
            pltpu.sync_copy(rows_v, acc_sh.at[dst_v.at[0]], add=True)

        plsc.subcore_barrier()
        _per_subcore_slice(sid, lambda lo, n: pltpu.sync_copy(
            acc_sh.at[pl.ds(lo, n)], out_hbm.at[cid, pl.ds(lo, n)]))

    return k(y, src, dst, zrows)


def _sc_gather(table, idx):
    nidx = idx.shape[0]
    width = table.shape[1]

    @functools.partial(
        pl.kernel,
        out_type=jax.ShapeDtypeStruct((nidx, width), table.dtype),
        mesh=_mesh(),
        scratch_types=[
            pltpu.VMEM((1, CHUNK), jnp.int32),
            pltpu.VMEM((CHUNK, width), table.dtype),
        ],
    )
    def k(t_hbm, i_hbm, o_hbm, idx_v, rows_v):
        cid = lax.axis_index("c")
        sid = lax.axis_index("s")
        wid = sid * 2 + cid

        @pl.loop(wid, nidx // CHUNK, step=NTILES)
        def _(g):
            base = g * CHUNK
            pltpu.sync_copy(i_hbm.at[pl.ds(base, CHUNK)], idx_v.at[0])
            pltpu.sync_copy(t_hbm.at[idx_v.at[0]], rows_v)
            pltpu.sync_copy(rows_v, o_hbm.at[pl.ds(base, CHUNK)])

    return k(table, idx)


_BN = 1000
_BE = 2000


def _tc_prep(deg2, x):
    def body(d0, d1, xb, dinv_ref, y1_ref):
        deg = d0[0][:, 0:1] + d1[0][:, 0:1] + 1.0
        dinv = lax.rsqrt(jnp.maximum(deg, 1.0))
        dinv_ref[...] = dinv
        y1_ref[...] = xb[...] * dinv

    return pl.pallas_call(
        body,
        grid=(N // _BN,),
        in_specs=[
            pl.BlockSpec((1, _BN, C), lambda i: (0, i, 0)),
            pl.BlockSpec((1, _BN, C), lambda i: (1, i, 0)),
            pl.BlockSpec((_BN, C), lambda i: (i, 0)),
        ],
        out_specs=[
            pl.BlockSpec((_BN, 1), lambda i: (i, 0)),
            pl.BlockSpec((_BN, C), lambda i: (i, 0)),
        ],
        out_shape=[
            jax.ShapeDtypeStruct((N, 1), jnp.float32),
            jax.ShapeDtypeStruct((N, C), jnp.float32),
        ],
    )(deg2, deg2, x)


def _tc_mid(p, y1, dinv, W1, b1, W2):
    def body(a0, a1, y1b, dv, w1, b1b, w2, y2_ref):
        a = (a0[0] + a1[0] + y1b[...]) * dv[...]
        z1 = jnp.maximum(
            jnp.dot(a, w1[...], preferred_element_type=jnp.float32) + b1b[...],
            0.0)
        h2 = jnp.dot(z1, w2[...], preferred_element_type=jnp.float32)
        y2_ref[...] = h2 * dv[...]

    return pl.pallas_call(
        body,
        grid=(N // _BN,),
        in_specs=[
            pl.BlockSpec((1, _BN, C), lambda i: (0, i, 0)),
            pl.BlockSpec((1, _BN, C), lambda i: (1, i, 0)),
            pl.BlockSpec((_BN, C), lambda i: (i, 0)),
            pl.BlockSpec((_BN, 1), lambda i: (i, 0)),
            pl.BlockSpec((C, 2 * C), lambda i: (0, 0)),
            pl.BlockSpec((1, 2 * C), lambda i: (0, 0)),
            pl.BlockSpec((2 * C, C), lambda i: (0, 0)),
        ],
        out_specs=pl.BlockSpec((_BN, C), lambda i: (i, 0)),
        out_shape=jax.ShapeDtypeStruct((N, C), jnp.float32),
    )(p, p, y1, dinv, W1, b1, W2)


def _tc_table(p, y2, dinv, b2, fc1Ws, fc1bs):
    def body(a0, a1, y2b, dv, b2b, w, bb, out_ref):
        z = (a0[0] + a1[0] + y2b[...]) * dv[...] + b2b[...]
        out_ref[0] = jnp.dot(z, w[0], preferred_element_type=jnp.float32) + bb[0]

    return pl.pallas_call(
        body,
        grid=(N // _BN, 2),
        in_specs=[
            pl.BlockSpec((1, _BN, C), lambda i, j: (0, i, 0)),
            pl.BlockSpec((1, _BN, C), lambda i, j: (1, i, 0)),
            pl.BlockSpec((_BN, C), lambda i, j: (i, 0)),
            pl.BlockSpec((_BN, 1), lambda i, j: (i, 0)),
            pl.BlockSpec((1, C), lambda i, j: (0, 0)),
            pl.BlockSpec((1, C, C), lambda i, j: (j, 0, 0)),
            pl.BlockSpec((1, 1, C), lambda i, j: (j, 0, 0)),
        ],
        out_specs=pl.BlockSpec((1, _BN, C), lambda i, j: (j, i, 0)),
        out_shape=jax.ShapeDtypeStruct((2, N, C), jnp.float32),
    )(p, p, y2, dinv, b2, fc1Ws, fc1bs)


def _tc_decoder(G, fc2_W, fc2_b, fc3_W, fc3_b, w4row, b4):
    ne = G.shape[0] // 2
    nb = ne // _BE

    def body(g0, g1, w2, b2b, w3, b3b, w4, b4b, o_ref):
        u = jnp.maximum(g0[...].astype(jnp.float32) +
                        g1[...].astype(jnp.float32), 0.0)
        h1 = jnp.maximum(
            jnp.dot(u, w2[...], preferred_element_type=jnp.float32) + b2b[...],
            0.0)
        h2 = jnp.maximum(
            jnp.dot(h1, w3[...], preferred_element_type=jnp.float32) + b3b[...],
            0.0)
        o_ref[...] = jnp.sum(h2 * w4[...], axis=1, keepdims=True) + b4b[...]

    return pl.pallas_call(
        body,
        grid=(nb,),
        in_specs=[
            pl.BlockSpec((_BE, C), lambda i: (i, 0)),
            pl.BlockSpec((_BE, C), lambda i: (i + nb, 0)),
            pl.BlockSpec((C, 64), lambda i: (0, 0)),
            pl.BlockSpec((1, 64), lambda i: (0, 0)),
            pl.BlockSpec((64, 32), lambda i: (0, 0)),
            pl.BlockSpec((1, 32), lambda i: (0, 0)),
            pl.BlockSpec((1, 32), lambda i: (0, 0)),
            pl.BlockSpec((1, 1), lambda i: (0, 0)),
        ],
        out_specs=pl.BlockSpec((_BE, 1), lambda i: (i, 0)),
        out_shape=jax.ShapeDtypeStruct((ne, 1), jnp.float32),
    )(G, G, fc2_W, fc2_b, fc3_W, fc3_b, w4row, b4)


def kernel(x, edge_index, W1, b1, W2, b2, fc1_W, fc1_b, fc2_W, fc2_b,
           fc3_W, fc3_b, fc4_W, fc4_b):
    e0 = edge_index[0].astype(jnp.int32)
    e1 = edge_index[1].astype(jnp.int32)

    ones_chunk = jnp.ones((CHUNK, C), jnp.float32)
    zerosNC = jnp.zeros((N, C), jnp.float32)

    deg2 = _sc_degree(e1, ones_chunk, zerosNC)
    dinv, y1 = _tc_prep(deg2, x)
    p1 = _sc_aggregate(y1, e0, e1, zerosNC)
    y2 = _tc_mid(p1, y1, dinv, W1, b1.reshape(1, -1), W2)
    p2 = _sc_aggregate(y2, e0, e1, zerosNC)

    fc1Ws = fc1_W.reshape(2, C, C)
    fc1bs = jnp.stack([fc1_b, jnp.zeros_like(fc1_b)]).reshape(2, 1, C)
    table = _tc_table(p2, y2, dinv, b2.reshape(1, -1), fc1Ws, fc1bs)

    t2 = table.reshape(2 * N, C)
    half = E // 2
    outs = []
    for s in range(2):
        sl = slice(s * half, (s + 1) * half)
        idx_h = jnp.concatenate([e0[sl], e1[sl] + N])
        G = _sc_gather(t2, idx_h)
        outs.append(_tc_decoder(G, fc2_W, fc2_b.reshape(1, -1), fc3_W,
                                fc3_b.reshape(1, -1), fc4_W.reshape(1, -1),
                                fc4_b.reshape(1, 1)))
    return jnp.concatenate(outs).reshape(E)


# speedup vs baseline: 2.3023x; 1.0328x over previous
import functools

import jax
import jax.numpy as jnp
from jax import lax
from jax.experimental import pallas as pl
from jax.experimental.pallas import tpu as pltpu
from jax.experimental.pallas import tpu_sc as plsc

N = 10000
E = 320000
C = 128
CHUNK = 128
NTILES = 32
RPS = 632
RPS_LAST = N - 15 * RPS


def _mesh():
    return plsc.VectorSubcoreMesh(core_axis_name="c", subcore_axis_name="s",
                                  num_cores=2)


def _per_subcore_slice(sid, fn):
    lo = pl.multiple_of(sid * RPS, 8)

    @pl.when(sid < 15)
    def _():
        fn(lo, RPS)

    @pl.when(sid == 15)
    def _():
        fn(lo, RPS_LAST)


DW = 16


def _sc_degree(dst, ones_chunk, zrows):
    nch = dst.shape[0] // CHUNK

    @functools.partial(
        pl.kernel,
        out_type=jax.ShapeDtypeStruct((2, N, DW), jnp.float32),
        mesh=_mesh(),
        compiler_params=pltpu.CompilerParams(use_tc_tiling_on_sc=False),
        scratch_types=[
            pltpu.VMEM((1, CHUNK), jnp.int32),
            pltpu.VMEM((CHUNK, DW), jnp.float32),
            pltpu.VMEM_SHARED((N, DW), jnp.float32),
        ],
    )
    def k(dst_hbm, ones_hbm, z_hbm, out_hbm, idx_v, ones_v, acc_sh):
        cid = lax.axis_index("c")
        sid = lax.axis_index("s")
        wid = sid * 2 + cid
        pltpu.sync_copy(ones_hbm, ones_v)
        _per_subcore_slice(sid, lambda lo, n: pltpu.sync_copy(
            z_hbm.at[pl.ds(lo, n)], acc_sh.at[pl.ds(lo, n)]))
        plsc.subcore_barrier()

        @pl.loop(wid, nch, step=NTILES)
        def _(g):
            pltpu.sync_copy(dst_hbm.at[pl.ds(g * CHUNK, CHUNK)], idx_v.at[0])
            pltpu.sync_copy(ones_v, acc_sh.at[idx_v.at[0]], add=True)

        plsc.subcore_barrier()
        _per_subcore_slice(sid, lambda lo, n: pltpu.sync_copy(
            acc_sh.at[pl.ds(lo, n)], out_hbm.at[cid, pl.ds(lo, n)]))

    return k(dst, ones_chunk, zrows)


SUP = 2
SUPE = SUP * CHUNK


def _sc_aggregate(y, src, dst, zrows):
    nch = src.shape[0] // CHUNK

    @functools.partial(
        pl.kernel,
        out_type=jax.ShapeDtypeStruct((2, N, C), jnp.float32),
        mesh=_mesh(),
        scratch_types=[
            pltpu.VMEM((1, CHUNK), jnp.int32),
            pltpu.VMEM((1, CHUNK), jnp.int32),
            pltpu.VMEM((CHUNK, C), jnp.float32),
            pltpu.VMEM_SHARED((N, C), jnp.float32),
        ],
    )
    def k(y_hbm, src_hbm, dst_hbm, z_hbm, out_hbm, src_v, dst_v, rows_v,
          acc_sh):
        cid = lax.axis_index("c")
        sid = lax.axis_index("s")
        wid = sid * 2 + cid
        _per_subcore_slice(sid, lambda lo, n: pltpu.sync_copy(
            z_hbm.at[pl.ds(lo, n)], acc_sh.at[pl.ds(lo, n)]))
        plsc.subcore_barrier()

        @pl.loop(wid, nch, step=NTILES)
        def _(g):
            base = g * CHUNK
            pltpu.sync_copy(src_hbm.at[pl.ds(base, CHUNK)], src_v.at[0])
            pltpu.sync_copy(dst_hbm.at[pl.ds(base, CHUNK)], dst_v.at[0])
            pltpu.sync_copy(y_hbm.at[src_v.at[0]], rows_v)
            pltpu.sync_copy(rows_v, acc_sh.at[dst_v.at[0]], add=True)

        plsc.subcore_barrier()
        _per_subcore_slice(sid, lambda lo, n: pltpu.sync_copy(
            acc_sh.at[pl.ds(lo, n)], out_hbm.at[cid, pl.ds(lo, n)]))

    return k(y, src, dst, zrows)


def _sc_gather(table, idx):
    nidx = idx.shape[0]
    width = table.shape[1]

    @functools.partial(
        pl.kernel,
        out_type=jax.ShapeDtypeStruct((nidx, width), table.dtype),
        mesh=_mesh(),
        scratch_types=[
            pltpu.VMEM((1, CHUNK), jnp.int32),
            pltpu.VMEM((CHUNK, width), table.dtype),
        ],
    )
    def k(t_hbm, i_hbm, o_hbm, idx_v, rows_v):
        cid = lax.axis_index("c")
        sid = lax.axis_index("s")
        wid = sid * 2 + cid

        @pl.loop(wid, nidx // CHUNK, step=NTILES)
        def _(g):
            base = g * CHUNK
            pltpu.sync_copy(i_hbm.at[pl.ds(base, CHUNK)], idx_v.at[0])
            pltpu.sync_copy(t_hbm.at[idx_v.at[0]], rows_v)
            pltpu.sync_copy(rows_v, o_hbm.at[pl.ds(base, CHUNK)])

    return k(table, idx)


_BN = 1000
_BE = 2000


def _tc_prep(deg2, x):
    def body(d0, d1, xb, dinv_ref, y1_ref):
        deg = d0[0][:, 0:1] + d1[0][:, 0:1] + 1.0
        dinv = lax.rsqrt(jnp.maximum(deg, 1.0))
        dinv_ref[...] = dinv
        y1_ref[...] = xb[...] * dinv

    return pl.pallas_call(
        body,
        grid=(N // _BN,),
        in_specs=[
            pl.BlockSpec((1, _BN, DW), lambda i: (0, i, 0)),
            pl.BlockSpec((1, _BN, DW), lambda i: (1, i, 0)),
            pl.BlockSpec((_BN, C), lambda i: (i, 0)),
        ],
        out_specs=[
            pl.BlockSpec((_BN, 1), lambda i: (i, 0)),
            pl.BlockSpec((_BN, C), lambda i: (i, 0)),
        ],
        out_shape=[
            jax.ShapeDtypeStruct((N, 1), jnp.float32),
            jax.ShapeDtypeStruct((N, C), jnp.float32),
        ],
    )(deg2, deg2, x)


def _tc_mid(p, y1, dinv, W1, b1, W2):
    def body(a0, a1, y1b, dv, w1, b1b, w2, y2_ref):
        a = (a0[0] + a1[0] + y1b[...]) * dv[...]
        z1 = jnp.maximum(
            jnp.dot(a, w1[...], preferred_element_type=jnp.float32) + b1b[...],
            0.0)
        h2 = jnp.dot(z1, w2[...], preferred_element_type=jnp.float32)
        y2_ref[...] = h2 * dv[...]

    return pl.pallas_call(
        body,
        grid=(N // _BN,),
        in_specs=[
            pl.BlockSpec((1, _BN, C), lambda i: (0, i, 0)),
            pl.BlockSpec((1, _BN, C), lambda i: (1, i, 0)),
            pl.BlockSpec((_BN, C), lambda i: (i, 0)),
            pl.BlockSpec((_BN, 1), lambda i: (i, 0)),
            pl.BlockSpec((C, 2 * C), lambda i: (0, 0)),
            pl.BlockSpec((1, 2 * C), lambda i: (0, 0)),
            pl.BlockSpec((2 * C, C), lambda i: (0, 0)),
        ],
        out_specs=pl.BlockSpec((_BN, C), lambda i: (i, 0)),
        out_shape=jax.ShapeDtypeStruct((N, C), jnp.float32),
    )(p, p, y1, dinv, W1, b1, W2)


def _tc_table(p, y2, dinv, b2, fc1Ws, fc1bs):
    def body(a0, a1, y2b, dv, b2b, w, bb, out_ref):
        z = (a0[0] + a1[0] + y2b[...]) * dv[...] + b2b[...]
        out_ref[0] = jnp.dot(z, w[0], preferred_element_type=jnp.float32) + bb[0]

    return pl.pallas_call(
        body,
        grid=(N // _BN, 2),
        in_specs=[
            pl.BlockSpec((1, _BN, C), lambda i, j: (0, i, 0)),
            pl.BlockSpec((1, _BN, C), lambda i, j: (1, i, 0)),
            pl.BlockSpec((_BN, C), lambda i, j: (i, 0)),
            pl.BlockSpec((_BN, 1), lambda i, j: (i, 0)),
            pl.BlockSpec((1, C), lambda i, j: (0, 0)),
            pl.BlockSpec((1, C, C), lambda i, j: (j, 0, 0)),
            pl.BlockSpec((1, 1, C), lambda i, j: (j, 0, 0)),
        ],
        out_specs=pl.BlockSpec((1, _BN, C), lambda i, j: (j, i, 0)),
        out_shape=jax.ShapeDtypeStruct((2, N, C), jnp.float32),
    )(p, p, y2, dinv, b2, fc1Ws, fc1bs)


def _tc_decoder(G, fc2_W, fc2_b, fc3_W, fc3_b, w4row, b4):
    ne = G.shape[0] // 2
    nb = ne // _BE

    def body(g0, g1, w2, b2b, w3, b3b, w4, b4b, o_ref):
        u = jnp.maximum(g0[...].astype(jnp.float32) +
                        g1[...].astype(jnp.float32), 0.0)
        h1 = jnp.maximum(
            jnp.dot(u, w2[...], preferred_element_type=jnp.float32) + b2b[...],
            0.0)
        h2 = jnp.maximum(
            jnp.dot(h1, w3[...], preferred_element_type=jnp.float32) + b3b[...],
            0.0)
        o_ref[...] = jnp.sum(h2 * w4[...], axis=1, keepdims=True) + b4b[...]

    return pl.pallas_call(
        body,
        grid=(nb,),
        in_specs=[
            pl.BlockSpec((_BE, C), lambda i: (i, 0)),
            pl.BlockSpec((_BE, C), lambda i: (i + nb, 0)),
            pl.BlockSpec((C, 64), lambda i: (0, 0)),
            pl.BlockSpec((1, 64), lambda i: (0, 0)),
            pl.BlockSpec((64, 32), lambda i: (0, 0)),
            pl.BlockSpec((1, 32), lambda i: (0, 0)),
            pl.BlockSpec((1, 32), lambda i: (0, 0)),
            pl.BlockSpec((1, 1), lambda i: (0, 0)),
        ],
        out_specs=pl.BlockSpec((_BE, 1), lambda i: (i, 0)),
        out_shape=jax.ShapeDtypeStruct((ne, 1), jnp.float32),
    )(G, G, fc2_W, fc2_b, fc3_W, fc3_b, w4row, b4)


def kernel(x, edge_index, W1, b1, W2, b2, fc1_W, fc1_b, fc2_W, fc2_b,
           fc3_W, fc3_b, fc4_W, fc4_b):
    e0 = edge_index[0].astype(jnp.int32)
    e1 = edge_index[1].astype(jnp.int32)

    ones_chunk = jnp.ones((CHUNK, DW), jnp.float32)
    zerosNC = jnp.zeros((N, C), jnp.float32)

    deg2 = _sc_degree(e1, ones_chunk, jnp.zeros((N, DW), jnp.float32))
    dinv, y1 = _tc_prep(deg2, x)
    p1 = _sc_aggregate(y1, e0, e1, zerosNC)
    y2 = _tc_mid(p1, y1, dinv, W1, b1.reshape(1, -1), W2)
    p2 = _sc_aggregate(y2, e0, e1, zerosNC)

    fc1Ws = fc1_W.reshape(2, C, C)
    fc1bs = jnp.stack([fc1_b, jnp.zeros_like(fc1_b)]).reshape(2, 1, C)
    table = _tc_table(p2, y2, dinv, b2.reshape(1, -1), fc1Ws, fc1bs)

    t2 = table.reshape(2 * N, C)
    half = E // 2
    outs = []
    for s in range(2):
        sl = slice(s * half, (s + 1) * half)
        idx_h = jnp.concatenate([e0[sl], e1[sl] + N])
        G = _sc_gather(t2, idx_h)
        outs.append(_tc_decoder(G, fc2_W, fc2_b.reshape(1, -1), fc3_W,
                                fc3_b.reshape(1, -1), fc4_W.reshape(1, -1),
                                fc4_b.reshape(1, 1)))
    return jnp.concatenate(outs).reshape(E)


# speedup vs baseline: 2.3495x; 1.0205x over previous
import functools

import jax
import jax.numpy as jnp
from jax import lax
from jax.experimental import pallas as pl
from jax.experimental.pallas import tpu as pltpu
from jax.experimental.pallas import tpu_sc as plsc

N = 10000
E = 320000
C = 128
CHUNK = 128
NTILES = 32
RPS = 632
RPS_LAST = N - 15 * RPS


def _mesh():
    return plsc.VectorSubcoreMesh(core_axis_name="c", subcore_axis_name="s",
                                  num_cores=2)


def _per_subcore_slice(sid, fn):
    lo = pl.multiple_of(sid * RPS, 8)

    @pl.when(sid < 15)
    def _():
        fn(lo, RPS)

    @pl.when(sid == 15)
    def _():
        fn(lo, RPS_LAST)


DW = 16


def _sc_degree(dst, ones_chunk, zrows):
    nch = dst.shape[0] // CHUNK

    @functools.partial(
        pl.kernel,
        out_type=jax.ShapeDtypeStruct((2, N, DW), jnp.float32),
        mesh=_mesh(),
        compiler_params=pltpu.CompilerParams(use_tc_tiling_on_sc=False),
        scratch_types=[
            pltpu.VMEM((1, CHUNK), jnp.int32),
            pltpu.VMEM((CHUNK, DW), jnp.float32),
            pltpu.VMEM_SHARED((N, DW), jnp.float32),
        ],
    )
    def k(dst_hbm, ones_hbm, z_hbm, out_hbm, idx_v, ones_v, acc_sh):
        cid = lax.axis_index("c")
        sid = lax.axis_index("s")
        wid = sid * 2 + cid
        pltpu.sync_copy(ones_hbm, ones_v)
        _per_subcore_slice(sid, lambda lo, n: pltpu.sync_copy(
            z_hbm.at[pl.ds(lo, n)], acc_sh.at[pl.ds(lo, n)]))
        plsc.subcore_barrier()

        @pl.loop(wid, nch, step=NTILES)
        def _(g):
            pltpu.sync_copy(dst_hbm.at[pl.ds(g * CHUNK, CHUNK)], idx_v.at[0])
            pltpu.sync_copy(ones_v, acc_sh.at[idx_v.at[0]], add=True)

        plsc.subcore_barrier()
        _per_subcore_slice(sid, lambda lo, n: pltpu.sync_copy(
            acc_sh.at[pl.ds(lo, n)], out_hbm.at[cid, pl.ds(lo, n)]))

    return k(dst, ones_chunk, zrows)


SUP = 2
SUPE = SUP * CHUNK


def _sc_aggregate(y, src, dst, zrows):
    nch = src.shape[0] // CHUNK

    @functools.partial(
        pl.kernel,
        out_type=jax.ShapeDtypeStruct((2, N, C), jnp.float32),
        mesh=_mesh(),
        scratch_types=[
            pltpu.VMEM((1, CHUNK), jnp.int32),
            pltpu.VMEM((1, CHUNK), jnp.int32),
            pltpu.VMEM((CHUNK, C), jnp.float32),
            pltpu.VMEM_SHARED((N, C), jnp.float32),
        ],
    )
    def k(y_hbm, src_hbm, dst_hbm, z_hbm, out_hbm, src_v, dst_v, rows_v,
          acc_sh):
        cid = lax.axis_index("c")
        sid = lax.axis_index("s")
        wid = sid * 2 + cid
        _per_subcore_slice(sid, lambda lo, n: pltpu.sync_copy(
            z_hbm.at[pl.ds(lo, n)], acc_sh.at[pl.ds(lo, n)]))
        plsc.subcore_barrier()

        @pl.loop(wid, nch, step=NTILES)
        def _(g):
            base = g * CHUNK
            pltpu.sync_copy(src_hbm.at[pl.ds(base, CHUNK)], src_v.at[0])
            pltpu.sync_copy(dst_hbm.at[pl.ds(base, CHUNK)], dst_v.at[0])
            pltpu.sync_copy(y_hbm.at[src_v.at[0]], rows_v)
            pltpu.sync_copy(rows_v, acc_sh.at[dst_v.at[0]], add=True)

        plsc.subcore_barrier()
        _per_subcore_slice(sid, lambda lo, n: pltpu.sync_copy(
            acc_sh.at[pl.ds(lo, n)], out_hbm.at[cid, pl.ds(lo, n)]))

    return k(y, src, dst, zrows)


def _sc_gather(table, idx, part, nparts, tok_a, tok_b):
    ech = E // CHUNK
    cpp = ech // nparts
    width = table.shape[1]

    @functools.partial(
        pl.kernel,
        out_type=jax.ShapeDtypeStruct((2 * cpp * CHUNK, width), table.dtype),
        mesh=_mesh(),
        scratch_types=[
            pltpu.VMEM((1, CHUNK), jnp.int32),
            pltpu.VMEM((CHUNK, width), table.dtype),
        ],
    )
    def k(t_hbm, i_hbm, tok_a_hbm, tok_b_hbm, o_hbm, idx_v, rows_v):
        del tok_a_hbm, tok_b_hbm
        cid = lax.axis_index("c")
        sid = lax.axis_index("s")
        wid = sid * 2 + cid

        @pl.loop(wid, 2 * cpp, step=NTILES)
        def _(g):
            src_chunk = jnp.where(g < cpp, part * cpp + g,
                                  ech + part * cpp + (g - cpp))
            pltpu.sync_copy(i_hbm.at[pl.ds(src_chunk * CHUNK, CHUNK)],
                            idx_v.at[0])
            pltpu.sync_copy(t_hbm.at[idx_v.at[0]], rows_v)
            pltpu.sync_copy(rows_v, o_hbm.at[pl.ds(g * CHUNK, CHUNK)])

    return k(table, idx, tok_a, tok_b)


_BN = 1000
_BE = 2000


def _tc_prep(deg2, x):
    def body(d0, d1, xb, dinv_ref, y1_ref):
        deg = d0[0][:, 0:1] + d1[0][:, 0:1] + 1.0
        dinv = lax.rsqrt(jnp.maximum(deg, 1.0))
        dinv_ref[...] = dinv
        y1_ref[...] = xb[...] * dinv

    return pl.pallas_call(
        body,
        grid=(N // _BN,),
        in_specs=[
            pl.BlockSpec((1, _BN, DW), lambda i: (0, i, 0)),
            pl.BlockSpec((1, _BN, DW), lambda i: (1, i, 0)),
            pl.BlockSpec((_BN, C), lambda i: (i, 0)),
        ],
        out_specs=[
            pl.BlockSpec((_BN, 1), lambda i: (i, 0)),
            pl.BlockSpec((_BN, C), lambda i: (i, 0)),
        ],
        out_shape=[
            jax.ShapeDtypeStruct((N, 1), jnp.float32),
            jax.ShapeDtypeStruct((N, C), jnp.float32),
        ],
    )(deg2, deg2, x)


def _tc_mid(p, y1, dinv, W1, b1, W2):
    def body(a0, a1, y1b, dv, w1, b1b, w2, y2_ref):
        a = (a0[0] + a1[0] + y1b[...]) * dv[...]
        z1 = jnp.maximum(
            jnp.dot(a, w1[...], preferred_element_type=jnp.float32) + b1b[...],
            0.0)
        h2 = jnp.dot(z1, w2[...], preferred_element_type=jnp.float32)
        y2_ref[...] = h2 * dv[...]

    return pl.pallas_call(
        body,
        grid=(N // _BN,),
        in_specs=[
            pl.BlockSpec((1, _BN, C), lambda i: (0, i, 0)),
            pl.BlockSpec((1, _BN, C), lambda i: (1, i, 0)),
            pl.BlockSpec((_BN, C), lambda i: (i, 0)),
            pl.BlockSpec((_BN, 1), lambda i: (i, 0)),
            pl.BlockSpec((C, 2 * C), lambda i: (0, 0)),
            pl.BlockSpec((1, 2 * C), lambda i: (0, 0)),
            pl.BlockSpec((2 * C, C), lambda i: (0, 0)),
        ],
        out_specs=pl.BlockSpec((_BN, C), lambda i: (i, 0)),
        out_shape=jax.ShapeDtypeStruct((N, C), jnp.float32),
    )(p, p, y1, dinv, W1, b1, W2)


def _tc_table(p, y2, dinv, b2, fc1Ws, fc1bs):
    def body(a0, a1, y2b, dv, b2b, w, bb, out_ref):
        z = (a0[0] + a1[0] + y2b[...]) * dv[...] + b2b[...]
        out_ref[0] = jnp.dot(z, w[0], preferred_element_type=jnp.float32) + bb[0]

    return pl.pallas_call(
        body,
        grid=(N // _BN, 2),
        in_specs=[
            pl.BlockSpec((1, _BN, C), lambda i, j: (0, i, 0)),
            pl.BlockSpec((1, _BN, C), lambda i, j: (1, i, 0)),
            pl.BlockSpec((_BN, C), lambda i, j: (i, 0)),
            pl.BlockSpec((_BN, 1), lambda i, j: (i, 0)),
            pl.BlockSpec((1, C), lambda i, j: (0, 0)),
            pl.BlockSpec((1, C, C), lambda i, j: (j, 0, 0)),
            pl.BlockSpec((1, 1, C), lambda i, j: (j, 0, 0)),
        ],
        out_specs=pl.BlockSpec((1, _BN, C), lambda i, j: (j, i, 0)),
        out_shape=jax.ShapeDtypeStruct((2, N, C), jnp.float32),
    )(p, p, y2, dinv, b2, fc1Ws, fc1bs)


def _tc_decoder(G, fc2_W, fc2_b, fc3_W, fc3_b, w4row, b4):
    ne = G.shape[0] // 2
    nb = ne // _BE

    def body(g0, g1, w2, b2b, w3, b3b, w4, b4b, o_ref):
        u = jnp.maximum(g0[...].astype(jnp.float32) +
                        g1[...].astype(jnp.float32), 0.0)
        h1 = jnp.maximum(
            jnp.dot(u, w2[...], preferred_element_type=jnp.float32) + b2b[...],
            0.0)
        h2 = jnp.maximum(
            jnp.dot(h1, w3[...], preferred_element_type=jnp.float32) + b3b[...],
            0.0)
        o_ref[...] = jnp.sum(h2 * w4[...], axis=1, keepdims=True) + b4b[...]

    return pl.pallas_call(
        body,
        grid=(nb,),
        in_specs=[
            pl.BlockSpec((_BE, C), lambda i: (i, 0)),
            pl.BlockSpec((_BE, C), lambda i: (i + nb, 0)),
            pl.BlockSpec((C, 64), lambda i: (0, 0)),
            pl.BlockSpec((1, 64), lambda i: (0, 0)),
            pl.BlockSpec((64, 32), lambda i: (0, 0)),
            pl.BlockSpec((1, 32), lambda i: (0, 0)),
            pl.BlockSpec((1, 32), lambda i: (0, 0)),
            pl.BlockSpec((1, 1), lambda i: (0, 0)),
        ],
        out_specs=pl.BlockSpec((_BE, 1), lambda i: (i, 0)),
        out_shape=jax.ShapeDtypeStruct((ne, 1), jnp.float32),
    )(G, G, fc2_W, fc2_b, fc3_W, fc3_b, w4row, b4)


def kernel(x, edge_index, W1, b1, W2, b2, fc1_W, fc1_b, fc2_W, fc2_b,
           fc3_W, fc3_b, fc4_W, fc4_b):
    e0 = edge_index[0].astype(jnp.int32)
    e1 = edge_index[1].astype(jnp.int32)

    ones_chunk = jnp.ones((CHUNK, DW), jnp.float32)
    zerosNC = jnp.zeros((N, C), jnp.float32)

    deg2 = _sc_degree(e1, ones_chunk, jnp.zeros((N, DW), jnp.float32))
    dinv, y1 = _tc_prep(deg2, x)
    p1 = _sc_aggregate(y1, e0, e1, zerosNC)
    y2 = _tc_mid(p1, y1, dinv, W1, b1.reshape(1, -1), W2)
    p2 = _sc_aggregate(y2, e0, e1, zerosNC)

    fc1Ws = fc1_W.reshape(2, C, C)
    fc1bs = jnp.stack([fc1_b, jnp.zeros_like(fc1_b)]).reshape(2, 1, C)
    table = _tc_table(p2, y2, dinv, b2.reshape(1, -1), fc1Ws, fc1bs)

    t2 = table.reshape(2 * N, C)
    idx_cat = jnp.concatenate([e0, e1 + N])
    nparts = 4
    outs = []
    G = t2
    for s in range(nparts):
        tok_b = outs[s - 2] if s >= 2 else t2
        G = _sc_gather(t2, idx_cat, s, nparts, G, tok_b)
        outs.append(_tc_decoder(G, fc2_W, fc2_b.reshape(1, -1), fc3_W,
                                fc3_b.reshape(1, -1), fc4_W.reshape(1, -1),
                                fc4_b.reshape(1, 1)))
    return jnp.concatenate(outs).reshape(E)
